# baseline (speedup 1.0000x reference)
import jax
import jax.numpy as jnp
from jax.experimental import pallas as pl

_DIM = 64


def _zero_fill(out_ref):
    out_ref[...] = jnp.zeros_like(out_ref)


def kernel(tgt, times):
    del times
    batch = tgt.shape[0]
    return pl.pallas_call(
        _zero_fill,
        out_shape=jax.ShapeDtypeStruct((batch, _DIM), jnp.float32),
    )()
